# Initial kernel scaffold; baseline (speedup 1.0000x reference)
import jax
import jax.numpy as jnp
from jax.experimental import pallas as pl


def kernel(company_ids, patent_ids, patent_neighbors, industry_neighbors, first_patentee_neighbors, ipc_neighbors, appdate_neighbors, patent_table, patentee_table, ipc_table, industry_table, appdate_table, W_agg, b_agg, W_fil, b_fil):
    raise NotImplementedError("write your pallas kernel here")


# speedup vs baseline: 1.8738x; 1.8738x over previous
import functools

import jax
import jax.numpy as jnp
from jax import lax
from jax.experimental import pallas as pl
from jax.experimental.pallas import tpu as pltpu
from jax.experimental.pallas import tpu_sc as plsc

D = 128
B = 4096
CHUNK = 128
NW = 32

_GATHERS = (
    ("cemb", 1),
    ("pemb", 1),
    ("patn", 32),
    ("ind", 4),
    ("fp", 2),
    ("ipc", 8),
    ("date", 1),
)


def _sc_gather_all(patent_table, patentee_table, ipc_table, industry_table,
                   appdate_table, idx_chunks):
    mesh = plsc.VectorSubcoreMesh(core_axis_name="c", subcore_axis_name="s")
    nch = {name: idx_chunks[name].shape[0] for name, _ in _GATHERS}
    out_types = [jax.ShapeDtypeStruct((nch[name], CHUNK, D), jnp.float32)
                 for name, _ in _GATHERS]

    @functools.partial(
        pl.kernel, mesh=mesh,
        out_type=out_types,
        scratch_types=[
            pltpu.VMEM((CHUNK,), jnp.int32),
            pltpu.VMEM((CHUNK, D), jnp.float32),
            pltpu.SemaphoreType.DMA,
        ],
    )
    def k(pat_t, pee_t, ipc_t, ind_t, date_t,
          cemb_i, pemb_i, patn_i, ind_i, fp_i, ipc_i, date_i,
          cemb_o, pemb_o, patn_o, ind_o, fp_o, ipc_o, date_o,
          idx_v, rows_v, sem):
        wid = lax.axis_index("s") * 2 + lax.axis_index("c")

        def do_gather(table_ref, idx_ref, out_ref, nchunks):
            per_w = nchunks // NW
            base = wid * per_w

            def body(i, carry):
                c = base + i
                pltpu.sync_copy(idx_ref.at[c], idx_v)
                pltpu.async_copy(table_ref.at[idx_v], rows_v, sem).wait()
                pltpu.sync_copy(rows_v, out_ref.at[c])
                return carry

            lax.fori_loop(0, per_w, body, 0)

        do_gather(pee_t, cemb_i, cemb_o, nch["cemb"])
        do_gather(pat_t, pemb_i, pemb_o, nch["pemb"])
        do_gather(pat_t, patn_i, patn_o, nch["patn"])
        do_gather(ind_t, ind_i, ind_o, nch["ind"])
        do_gather(pee_t, fp_i, fp_o, nch["fp"])
        do_gather(ipc_t, ipc_i, ipc_o, nch["ipc"])
        do_gather(date_t, date_i, date_o, nch["date"])

    outs = k(patent_table, patentee_table, ipc_table, industry_table,
             appdate_table, *[idx_chunks[name] for name, _ in _GATHERS])
    return {name: o for (name, _), o in zip(_GATHERS, outs)}


def _compute_body(cemb_ref, pemb_ref, patn_ref, ind_ref, fp_ref, ipc_ref,
                  date_ref, wagg_ref, bagg_ref, wfil_ref, bfil_ref, out_ref):
    wa1 = wagg_ref[:D, :]
    wa2 = wagg_ref[D:, :]
    wf1 = wfil_ref[:D, :]
    wf2 = wfil_ref[D:, :]
    bagg = bagg_ref[...]
    bfil = bfil_ref[...]

    def side(center, groups, n_total):
        bb = center.shape[0]
        c_w = jnp.dot(center, wa1, preferred_element_type=jnp.float32)
        gsum = jnp.zeros((bb, D), jnp.float32)
        ssum = jnp.zeros((bb, D), jnp.float32)
        for rows, n in groups:
            h = jnp.dot(rows, wa2, preferred_element_type=jnp.float32)
            h3 = h.reshape(bb, n, D)
            r3 = rows.reshape(bb, n, D)
            gate = jax.nn.sigmoid(h3 + c_w[:, None, :] + bagg[None])
            gsum = gsum + jnp.sum(r3 * gate, axis=1)
            ssum = ssum + jnp.sum(r3, axis=1)
        agg = gsum * (1.0 / n_total)
        avg = ssum * (1.0 / n_total)
        fg = jax.nn.sigmoid(
            jnp.dot(center, wf1, preferred_element_type=jnp.float32)
            + jnp.dot(avg, wf2, preferred_element_type=jnp.float32) + bfil)
        x = center * (1.0 - fg) + agg
        return jnp.where(x >= 0, x, 0.2 * x)

    cemb = cemb_ref[...]
    pemb = pemb_ref[...]
    cs = side(cemb, [(ind_ref[...], 4), (patn_ref[...], 32)], 36.0)
    ps = side(pemb, [(fp_ref[...], 2), (ipc_ref[...], 8), (date_ref[...], 1)],
              11.0)
    out_ref[...] = jax.nn.sigmoid(jnp.sum(cs * ps, axis=1, keepdims=True))


def _tc_compute(cemb, pemb, patn, ind, fp, ipc, date, W_agg, b_agg, W_fil,
                b_fil, bb=256, interpret=False):
    nblk = B // bb

    def row_spec(n):
        return pl.BlockSpec((bb * n, D), lambda i: (i, 0))

    full = lambda shape: pl.BlockSpec(shape, lambda i: (0, 0))
    out = pl.pallas_call(
        _compute_body,
        grid=(nblk,),
        in_specs=[
            row_spec(1), row_spec(1), row_spec(32), row_spec(4), row_spec(2),
            row_spec(8), row_spec(1),
            full((2 * D, D)), full((1, D)), full((2 * D, D)), full((1, D)),
        ],
        out_specs=pl.BlockSpec((bb, 1), lambda i: (i, 0)),
        out_shape=jax.ShapeDtypeStruct((B, 1), jnp.float32),
        interpret=interpret,
    )(cemb, pemb, patn, ind, fp, ipc, date, W_agg, b_agg.reshape(1, D),
      W_fil, b_fil.reshape(1, D))
    return out.reshape(B)


def kernel(company_ids, patent_ids, patent_neighbors, industry_neighbors,
           first_patentee_neighbors, ipc_neighbors, appdate_neighbors,
           patent_table, patentee_table, ipc_table, industry_table,
           appdate_table, W_agg, b_agg, W_fil, b_fil):
    idx_flat = {
        "cemb": company_ids,
        "pemb": patent_ids,
        "patn": patent_neighbors,
        "ind": industry_neighbors,
        "fp": first_patentee_neighbors,
        "ipc": ipc_neighbors,
        "date": appdate_neighbors,
    }
    idx_chunks = {
        name: a.astype(jnp.int32).reshape(-1, CHUNK)
        for name, a in idx_flat.items()
    }
    g = _sc_gather_all(patent_table, patentee_table, ipc_table,
                       industry_table, appdate_table, idx_chunks)
    rows = {name: v.reshape(-1, D) for name, v in g.items()}
    return _tc_compute(rows["cemb"], rows["pemb"], rows["patn"], rows["ind"],
                       rows["fp"], rows["ipc"], rows["date"],
                       W_agg, b_agg, W_fil, b_fil)
